# speedup vs baseline: 2.5435x; 2.5435x over previous
import functools

import jax
import jax.numpy as jnp
from jax import lax
from jax.experimental import pallas as pl
from jax.experimental.pallas import tpu as pltpu
from jax.experimental.pallas import tpu_sc as plsc

N, L, D = 4, 8192, 768
H = D // 4
KEEP = L // 2
SB, NB = 512, 16
NW = 32
PERW = N * KEEP // NW
CHUNK = 128

_mesh = plsc.VectorSubcoreMesh(core_axis_name="c", subcore_axis_name="s")


def _score_mm(a1, w3, b3):
    def body(x_ref, w_ref, b_ref, o_ref):
        o_ref[...] = jnp.dot(x_ref[...], w_ref[...], precision=None) + b_ref[...]

    blk = 2048
    return pl.pallas_call(
        body,
        grid=(N * L // blk,),
        in_specs=[pl.BlockSpec((blk, H), lambda i: (i, 0)),
                  pl.BlockSpec((H, 1), lambda i: (0, 0)),
                  pl.BlockSpec((1,), lambda i: (0,))],
        out_specs=pl.BlockSpec((blk, 1), lambda i: (i, 0)),
        out_shape=jax.ShapeDtypeStruct((N * L, 1), jnp.float32),
    )(a1, w3, b3)


def _rank_kernel(v3):
    def body(v_ref, o_ref, vs_ref):
        v = v_ref[0]
        vs_ref[...] = v
        k = lax.bitcast_convert_type(v, jnp.int32)
        kT = jnp.transpose(k.reshape(NB, SB))
        iota_col = lax.broadcasted_iota(jnp.int32, (SB, 1), 0)
        iota_row = lax.broadcasted_iota(jnp.int32, (1, SB), 1)
        cols = []
        for ib in range(NB):
            ki = kT[:, ib:ib + 1]
            ii = iota_col + ib * SB

            def body_ge(jb, acc):
                kj = lax.bitcast_convert_type(vs_ref[:, pl.ds(jb * SB, SB)], jnp.int32)
                return acc + (kj >= ki).astype(jnp.int32)

            def body_gt(jb, acc):
                kj = lax.bitcast_convert_type(vs_ref[:, pl.ds(jb * SB, SB)], jnp.int32)
                return acc + (kj > ki).astype(jnp.int32)

            acc = jnp.zeros((SB, SB), jnp.int32)
            acc = lax.fori_loop(0, ib, body_ge, acc)
            acc = lax.fori_loop(ib + 1, NB, body_gt, acc)
            kj = lax.bitcast_convert_type(vs_ref[:, pl.ds(ib * SB, SB)], jnp.int32)
            jj = iota_row + ib * SB
            dg = (kj > ki) | ((kj == ki) & (jj < ii))
            acc = acc + dg.astype(jnp.int32)
            cols.append(jnp.sum(acc, axis=1, keepdims=True))
        o_ref[0] = jnp.concatenate(cols, axis=1)

    return pl.pallas_call(
        body,
        grid=(N,),
        in_specs=[pl.BlockSpec((1, 1, L), lambda i: (i, 0, 0))],
        out_specs=pl.BlockSpec((1, SB, NB), lambda i: (i, 0, 0)),
        out_shape=jax.ShapeDtypeStruct((N, SB, NB), jnp.int32),
        scratch_shapes=[pltpu.VMEM((1, L), jnp.float32)],
    )(v3)


@functools.partial(
    pl.kernel, mesh=_mesh,
    out_type=jax.ShapeDtypeStruct((N * KEEP,), jnp.int32),
    scratch_types=[pltpu.VMEM((L,), jnp.int32), pltpu.VMEM((KEEP,), jnp.int32)],
    compiler_params=pltpu.CompilerParams(needs_layout_passes=False),
)
def _sc_invert(ranks_hbm, gids_hbm, r_v, ids_v):
    wid = lax.axis_index("s") * 2 + lax.axis_index("c")

    @pl.when(wid < N)
    def _():
        n = wid
        pltpu.sync_copy(ranks_hbm.at[n], r_v)

        def body(c, carry):
            rv = r_v[pl.ds(c * 16, 16)]
            vals = lax.broadcasted_iota(jnp.int32, (16,), 0) * SB + c + n * L
            mask = rv < KEEP
            plsc.store_scatter(ids_v, [jnp.minimum(rv, KEEP - 1)], vals, mask=mask)
            return carry

        lax.fori_loop(0, L // 16, body, 0)
        pltpu.sync_copy(ids_v, gids_hbm.at[pl.ds(n * KEEP, KEEP)])


@functools.partial(
    pl.kernel, mesh=_mesh,
    out_type=(jax.ShapeDtypeStruct((N * KEEP, D), jnp.float32),
              jax.ShapeDtypeStruct((N * KEEP, D), jnp.float32)),
    scratch_types=[pltpu.VMEM((PERW,), jnp.int32),
                   pltpu.VMEM((CHUNK, D), jnp.float32),
                   pltpu.SemaphoreType.DMA],
)
def _sc_gather(inp_hbm, pos_hbm, gids_hbm, oi_hbm, op_hbm, idx_v, rows_v, sem):
    wid = lax.axis_index("s") * 2 + lax.axis_index("c")
    base = wid * PERW
    pltpu.sync_copy(gids_hbm.at[pl.ds(base, PERW)], idx_v)
    for c in range(PERW // CHUNK):
        pltpu.async_copy(inp_hbm.at[idx_v.at[pl.ds(c * CHUNK, CHUNK)]], rows_v, sem).wait()
        pltpu.sync_copy(rows_v, oi_hbm.at[pl.ds(base + c * CHUNK, CHUNK)])
    for c in range(PERW // CHUNK):
        pltpu.async_copy(pos_hbm.at[idx_v.at[pl.ds(c * CHUNK, CHUNK)]], rows_v, sem).wait()
        pltpu.sync_copy(rows_v, op_hbm.at[pl.ds(base + c * CHUNK, CHUNK)])


def kernel(input, positin_embedding, W0, b0, W1, b1, W3, b3):
    gelu = functools.partial(jax.nn.gelu, approximate=False)
    a1 = gelu(jnp.dot(gelu(jnp.dot(input, W0) + b0), W1) + b1)
    sc = _score_mm(a1.reshape(N * L, H), W3, b3).reshape(N, L)
    v = jax.nn.softmax(sc, axis=-1)
    ranks = _rank_kernel(v[:, None, :])
    gids = _sc_invert(ranks.reshape(N, L))
    oi, op = _sc_gather(input.reshape(N * L, D), positin_embedding.reshape(N * L, D), gids)
    return oi.reshape(N, KEEP, D), op.reshape(N, KEEP, D)


# speedup vs baseline: 3.0068x; 1.1822x over previous
import functools

import jax
import jax.numpy as jnp
from jax import lax
from jax.experimental import pallas as pl
from jax.experimental.pallas import tpu as pltpu
from jax.experimental.pallas import tpu_sc as plsc

N, L, D = 4, 8192, 768
H = D // 4
KEEP = L // 2
LOG = 13
NW = 32
PERW = N * KEEP // NW
CHUNK = 128

_mesh = plsc.VectorSubcoreMesh(core_axis_name="c", subcore_axis_name="s")


def _score_mm(a1, w3, b3):
    def body(x_ref, w_ref, b_ref, o_ref):
        o_ref[...] = jnp.dot(x_ref[...], w_ref[...], precision=None) + b_ref[...]

    blk = 2048
    return pl.pallas_call(
        body,
        grid=(N * L // blk,),
        in_specs=[pl.BlockSpec((blk, H), lambda i: (i, 0)),
                  pl.BlockSpec((H, 1), lambda i: (0, 0)),
                  pl.BlockSpec((1,), lambda i: (0,))],
        out_specs=pl.BlockSpec((blk, 1), lambda i: (i, 0)),
        out_shape=jax.ShapeDtypeStruct((N * L, 1), jnp.float32),
    )(a1, w3, b3)


def _bitonic_ids(v3):
    def body(v_ref, o_ref):
        v = v_ref[0]
        k = lax.bitcast_convert_type(v, jnp.int32)
        ix = lax.broadcasted_iota(jnp.int32, (1, L), 1)
        pos = lax.broadcasted_iota(jnp.int32, (1, L), 1)
        for ph in range(1, LOG + 1):
            for j in range(ph - 1, -1, -1):
                d = 1 << j
                kp_m = pltpu.roll(k, L - d, 1)
                kp_p = pltpu.roll(k, d, 1)
                ip_m = pltpu.roll(ix, L - d, 1)
                ip_p = pltpu.roll(ix, d, 1)
                rh = ((pos >> j) & 1) == 1
                kb = jnp.where(rh, kp_p, kp_m)
                ib = jnp.where(rh, ip_p, ip_m)
                q = (k > kb) | ((k == kb) & (ix < ib))
                asc = ((pos >> ph) & 1) == 1
                keep = q ^ rh ^ asc
                k = jnp.where(keep, k, kb)
                ix = jnp.where(keep, ix, ib)
        o_ref[0] = ix

    return pl.pallas_call(
        body,
        grid=(N,),
        in_specs=[pl.BlockSpec((1, 1, L), lambda i: (i, 0, 0))],
        out_specs=pl.BlockSpec((1, 1, L), lambda i: (i, 0, 0)),
        out_shape=jax.ShapeDtypeStruct((N, 1, L), jnp.int32),
    )(v3)


@functools.partial(
    pl.kernel, mesh=_mesh,
    out_type=(jax.ShapeDtypeStruct((N * KEEP, D), jnp.float32),
              jax.ShapeDtypeStruct((N * KEEP, D), jnp.float32)),
    scratch_types=[pltpu.VMEM((PERW,), jnp.int32),
                   pltpu.VMEM((CHUNK, D), jnp.float32),
                   pltpu.SemaphoreType.DMA],
)
def _sc_gather(inp_hbm, pos_hbm, gids_hbm, oi_hbm, op_hbm, idx_v, rows_v, sem):
    wid = lax.axis_index("s") * 2 + lax.axis_index("c")
    base = wid * PERW
    pltpu.sync_copy(gids_hbm.at[pl.ds(base, PERW)], idx_v)
    for c in range(PERW // CHUNK):
        pltpu.async_copy(inp_hbm.at[idx_v.at[pl.ds(c * CHUNK, CHUNK)]], rows_v, sem).wait()
        pltpu.sync_copy(rows_v, oi_hbm.at[pl.ds(base + c * CHUNK, CHUNK)])
    for c in range(PERW // CHUNK):
        pltpu.async_copy(pos_hbm.at[idx_v.at[pl.ds(c * CHUNK, CHUNK)]], rows_v, sem).wait()
        pltpu.sync_copy(rows_v, op_hbm.at[pl.ds(base + c * CHUNK, CHUNK)])


def kernel(input, positin_embedding, W0, b0, W1, b1, W3, b3):
    gelu = functools.partial(jax.nn.gelu, approximate=False)
    a1 = gelu(jnp.dot(gelu(jnp.dot(input, W0) + b0), W1) + b1)
    sc = _score_mm(a1.reshape(N * L, H), W3, b3).reshape(N, L)
    v = jax.nn.softmax(sc, axis=-1)
    ids = _bitonic_ids(v[:, None, :])[:, 0, :KEEP]
    gids = (ids + jnp.arange(N, dtype=jnp.int32)[:, None] * L).reshape(-1)
    oi, op = _sc_gather(input.reshape(N * L, D), positin_embedding.reshape(N * L, D), gids)
    return oi.reshape(N, KEEP, D), op.reshape(N, KEEP, D)
